# Initial kernel scaffold; baseline (speedup 1.0000x reference)
import jax
import jax.numpy as jnp
from jax.experimental import pallas as pl


def kernel(drowsiness_index, gesture_sequence):
    raise NotImplementedError("write your pallas kernel here")


# speedup vs baseline: 1.4114x; 1.4114x over previous
import jax
import jax.numpy as jnp
from jax.experimental import pallas as pl

_MIN_STREAK_HIGH = 4
_MIN_STREAK_LOW = 7
_MIN_STREAKS_HIGH_ACT = 2
_MIN_STREAKS_LOW_ACT = 3
_HIGH_IMPACT_INITIAL = 0.18
_LOW_IMPACT_INITIAL = 0.05
_MAX_ADJUSTMENT = 0.35
_HIGH_DECAY = 0.5
_LOW_DECAY = 0.5


def _body(drows_ref, g_ref, out_ref):
    g = g_ref[...]
    B, T = g.shape
    on = (g == 2).astype(jnp.int32)
    col = jax.lax.broadcasted_iota(jnp.int32, (B, T), 1)

    prev = jnp.where(col == 0, 0, jnp.roll(on, 1, axis=1))
    start = on * (1 - prev)

    def shifted(k):
        return jnp.where(col < T - k, jnp.roll(on, -k, axis=1), 0)

    w = start
    win = on
    for k in range(1, _MIN_STREAK_LOW):
        win = win * shifted(k)
        if k == _MIN_STREAK_HIGH - 1:
            w4 = start * win
        if k == _MIN_STREAK_LOW - 1:
            w7 = start * win

    high = jnp.sum(w4, axis=1, keepdims=True)
    low = jnp.sum(w7, axis=1, keepdims=True)

    high_f = high.astype(jnp.float32)
    low_f = low.astype(jnp.float32)
    ha = _HIGH_IMPACT_INITIAL * jnp.exp(-_HIGH_DECAY * (high_f - _MIN_STREAKS_HIGH_ACT))
    ha = jnp.where(high >= _MIN_STREAKS_HIGH_ACT, ha, 0.0)
    la = _LOW_IMPACT_INITIAL * jnp.exp(-_LOW_DECAY * (low_f - _MIN_STREAKS_LOW_ACT))
    la = jnp.where(low >= _MIN_STREAKS_LOW_ACT, la, 0.0)
    adj = jnp.minimum(ha + la, _MAX_ADJUSTMENT)

    out_ref[...] = jnp.clip(drows_ref[...] + adj, 0.0, 1.0)


def kernel(drowsiness_index, gesture_sequence):
    gestures = jnp.squeeze(gesture_sequence, axis=-1)
    B, T = gestures.shape
    out = pl.pallas_call(
        _body,
        out_shape=jax.ShapeDtypeStruct((B, 1), jnp.float32),
    )(drowsiness_index, gestures)
    return out
